# Initial kernel scaffold; baseline (speedup 1.0000x reference)
import jax
import jax.numpy as jnp
from jax.experimental import pallas as pl


def kernel(node_attr_prime, edge_out_bar, u, batch, W1, b1, W2, b2):
    raise NotImplementedError("write your pallas kernel here")


# speedup vs baseline: 2.1806x; 2.1806x over previous
import jax
import jax.numpy as jnp
from jax import lax
from jax.experimental import pallas as pl
from jax.experimental.pallas import tpu as pltpu
from jax.experimental.pallas import tpu_sc as plsc

N = 100000
G = 512
DN = 256
DE = 16
CW = 16
HC = 128

NC = 2
NS = 16
NW = NC * NS

BLK = 128

_G64 = N // 64
_Q, _R = divmod(_G64, NW)
ROWS_BIG = 64 * (_Q + 1)
ROWS_SMALL = 64 * _Q
FULL_BLOCKS = ROWS_SMALL // BLK
TAIL_BIG = ROWS_BIG - FULL_BLOCKS * BLK
EXTRA = N - NW * ROWS_SMALL - _R * 64
EXTRA_OFF = N - EXTRA


def _zero_vmem(ref, rows, width):
    z = jnp.zeros((16,), jnp.float32)

    def body(r, carry):
        for j in range(width // 16):
            ref[r, pl.ds(j * 16, 16)] = z
        return carry

    lax.fori_loop(0, rows, body, 0)


def _seg_sum_kernel(node_hbm, edge_hbm, batch_hbm,
                    node_out, ec_out,
                    node_v, edge_v, idx_v, idx_tb, idx_ts,
                    acc_n, acc_ec):
    c = lax.axis_index("c")
    s = lax.axis_index("s")
    wid = c * NS + s

    base = wid * ROWS_SMALL + 64 * jnp.minimum(wid, _R)
    toff = base + FULL_BLOCKS * BLK

    for p in range(2):
        _zero_vmem(acc_n, G, HC)
        if p == 0:
            _zero_vmem(acc_ec, G // 4, HC)

        iota16 = lax.iota(jnp.int32, 16)
        ones16 = jnp.ones((16,), jnp.float32)

        def block_body(off, idx_ref, nrows):
            pltpu.sync_copy(batch_hbm.at[pl.ds(off, nrows)], idx_ref)
            pltpu.sync_copy(node_hbm.at[pl.ds(off, nrows), pl.ds(p * HC, HC)],
                            node_v.at[pl.ds(0, nrows)])
            if p == 0:
                eoff = pl.multiple_of(off // 8, 8)
                pltpu.sync_copy(edge_hbm.at[pl.ds(eoff, nrows // 8)],
                                edge_v.at[pl.ds(0, nrows // 8)])

            def gbody(g, carry):
                ids = idx_ref[pl.ds(g * 16, 16)]
                for r16 in range(16):
                    r = g * 16 + r16
                    seg = ids.at[jnp.full((16,), r16, jnp.int32)].get(
                        mode="promise_in_bounds")
                    for j in range(HC // 16):
                        v = node_v[r, pl.ds(j * 16, 16)]
                        plsc.addupdate_scatter(acc_n,
                                               [seg, iota16 + (j * 16)], v)
                    if p == 0:
                        ev = edge_v[2 * g + r16 // 8,
                                    pl.ds((r16 % 8) * 16, 16)]
                        erow = lax.shift_right_logical(seg, 2)
                        ecol = lax.shift_left(jnp.bitwise_and(seg, 3), 5)
                        plsc.addupdate_scatter(acc_ec, [erow, ecol + iota16],
                                               ev)
                        plsc.addupdate_scatter(acc_ec,
                                               [erow, ecol + 16 + iota16],
                                               ones16)
                return carry

            lax.fori_loop(0, nrows // 16, gbody, 0)

            rem = nrows - (nrows // 16) * 16
            if rem:
                ids = idx_ref[pl.ds(nrows - 16, 16)]
                for r16 in range(16 - rem, 16):
                    r = nrows - 16 + r16
                    seg = ids.at[jnp.full((16,), r16, jnp.int32)].get(
                        mode="promise_in_bounds")
                    for j in range(HC // 16):
                        v = node_v[r, pl.ds(j * 16, 16)]
                        plsc.addupdate_scatter(acc_n,
                                               [seg, iota16 + (j * 16)], v)
                    if p == 0:
                        ev = edge_v[r // 8, pl.ds((r % 8) * 16, 16)]
                        erow = lax.shift_right_logical(seg, 2)
                        ecol = lax.shift_left(jnp.bitwise_and(seg, 3), 5)
                        plsc.addupdate_scatter(acc_ec, [erow, ecol + iota16],
                                               ev)
                        plsc.addupdate_scatter(acc_ec,
                                               [erow, ecol + 16 + iota16],
                                               ones16)

        def block(b, carry):
            block_body(base + b * BLK, idx_v, BLK)
            return carry

        lax.fori_loop(0, FULL_BLOCKS, block, 0)

        @pl.when(wid < _R)
        def _tail_big():
            block_body(toff, idx_tb, TAIL_BIG)

        @pl.when(wid == NW - 1)
        def _tail_extra():
            block_body(EXTRA_OFF, idx_ts, EXTRA)

        pltpu.sync_copy(acc_n, node_out.at[wid, :, pl.ds(p * HC, HC)])
        if p == 0:
            pltpu.sync_copy(acc_ec, ec_out.at[wid])


def _segment_sums(node_attr_prime, edge_out_bar, batch):
    mesh = plsc.VectorSubcoreMesh(core_axis_name="c", subcore_axis_name="s",
                                  num_cores=NC, num_subcores=NS)
    f = pl.kernel(
        _seg_sum_kernel,
        out_type=[
            jax.ShapeDtypeStruct((NW, G, DN), jnp.float32),
            jax.ShapeDtypeStruct((NW, G // 4, HC), jnp.float32),
        ],
        mesh=mesh,
        scratch_types=[
            pltpu.VMEM((BLK, HC), jnp.float32),
            pltpu.VMEM((BLK // 8, HC), jnp.float32),
            pltpu.VMEM((BLK,), jnp.int32),
            pltpu.VMEM((TAIL_BIG,), jnp.int32),
            pltpu.VMEM((EXTRA,), jnp.int32),
            pltpu.VMEM((G, HC), jnp.float32),
            pltpu.VMEM((G // 4, HC), jnp.float32),
        ],
        compiler_params=pltpu.CompilerParams(needs_layout_passes=False),
    )
    edge_folded = edge_out_bar.reshape(N // 8, 8 * DE)
    return f(node_attr_prime, edge_folded, batch)


def _mlp_kernel(u_ref, np_ref, ec_ref, w1_ref, b1_ref, w2_ref,
                b2_ref, out_ref):
    nsum = jnp.sum(np_ref[...], axis=0)
    ec = jnp.sum(ec_ref[...], axis=0)
    esum = ec[:, 0:DE]
    cnt = ec[:, DE:2 * DE]
    inv = 1.0 / jnp.maximum(cnt[:, 0:1], 1.0)
    nmean = nsum * inv
    emean = esum * inv
    h = jnp.dot(u_ref[...], w1_ref[0:32, :],
                preferred_element_type=jnp.float32)
    h = h + jnp.dot(nmean, w1_ref[32:288, :],
                    preferred_element_type=jnp.float32)
    h = h + jnp.dot(emean, w1_ref[288:304, :],
                    preferred_element_type=jnp.float32)
    h = jnp.maximum(h + b1_ref[...], 0.0)
    o = jnp.dot(h, w2_ref[...], preferred_element_type=jnp.float32)
    out_ref[...] = jnp.maximum(o + b2_ref[...], 0.0)


def kernel(node_attr_prime, edge_out_bar, u, batch, W1, b1, W2, b2):
    node_part, ec_part = _segment_sums(node_attr_prime, edge_out_bar, batch)
    ec_part = ec_part.reshape(NW, G, 2 * DE)
    return pl.pallas_call(
        _mlp_kernel,
        out_shape=jax.ShapeDtypeStruct((G, 1), jnp.float32),
    )(u, node_part, ec_part, W1, b1.reshape(1, -1), W2,
      b2.reshape(1, 1))


# speedup vs baseline: 3.1816x; 1.4590x over previous
import jax
import jax.numpy as jnp
from jax import lax
from jax.experimental import pallas as pl
from jax.experimental.pallas import tpu as pltpu
from jax.experimental.pallas import tpu_sc as plsc

N = 100000
G = 512
DN = 256
DE = 16
CW = 16
HC = 128

NC = 2
NS = 16
NW = NC * NS

BLK = 128

_G64 = N // 64
_Q, _R = divmod(_G64, NW)
ROWS_BIG = 64 * (_Q + 1)
ROWS_SMALL = 64 * _Q
FULL_BLOCKS = ROWS_SMALL // BLK
TAIL_BIG = ROWS_BIG - FULL_BLOCKS * BLK
EXTRA = N - NW * ROWS_SMALL - _R * 64
EXTRA_OFF = N - EXTRA


def _zero_vmem(ref, rows, width):
    z = jnp.zeros((16,), jnp.float32)

    def body(r, carry):
        for j in range(width // 16):
            ref[r, pl.ds(j * 16, 16)] = z
        return carry

    lax.fori_loop(0, rows, body, 0)


def _seg_sum_kernel(node_hbm, edge_hbm, batch_hbm,
                    node_out, ec_out,
                    node_v, edge_v, idx_v, idx_tb, idx_ts,
                    acc_n, acc_ec):
    c = lax.axis_index("c")
    s = lax.axis_index("s")
    wid = c * NS + s

    base = wid * ROWS_SMALL + 64 * jnp.minimum(wid, _R)
    toff = base + FULL_BLOCKS * BLK

    for p in range(2):
        _zero_vmem(acc_n, G, HC)
        if p == 0:
            _zero_vmem(acc_ec, G // 4, HC)

        iota16 = lax.iota(jnp.int32, 16)
        ones16 = jnp.ones((16,), jnp.float32)

        def block_body(off, idx_ref, nrows):
            pltpu.sync_copy(batch_hbm.at[pl.ds(off, nrows)], idx_ref)
            pltpu.sync_copy(node_hbm.at[pl.ds(off, nrows), pl.ds(p * HC, HC)],
                            node_v.at[pl.ds(0, nrows)])
            if p == 0:
                eoff = pl.multiple_of(off // 8, 8)
                pltpu.sync_copy(edge_hbm.at[pl.ds(eoff, nrows // 8)],
                                edge_v.at[pl.ds(0, nrows // 8)])

            def gbody(g, carry):
                ids = idx_ref[pl.ds(g * 16, 16)]
                for r16 in range(16):
                    r = g * 16 + r16
                    vs = [node_v[r, pl.ds(j * 16, 16)]
                          for j in range(HC // 16)]
                    seg = ids.at[jnp.full((16,), r16, jnp.int32)].get(
                        mode="promise_in_bounds")
                    if p == 0:
                        ev = edge_v[2 * g + r16 // 8,
                                    pl.ds((r16 % 8) * 16, 16)]
                        erow = lax.shift_right_logical(seg, 2)
                        ecol = lax.shift_left(jnp.bitwise_and(seg, 3), 5)
                    for j in range(HC // 16):
                        plsc.addupdate_scatter(acc_n,
                                               [seg, iota16 + (j * 16)],
                                               vs[j])
                    if p == 0:
                        plsc.addupdate_scatter(acc_ec, [erow, ecol + iota16],
                                               ev)
                        plsc.addupdate_scatter(acc_ec,
                                               [erow, ecol + 16 + iota16],
                                               ones16)
                return carry

            lax.fori_loop(0, nrows // 16, gbody, 0)

            rem = nrows - (nrows // 16) * 16
            if rem:
                ids = idx_ref[pl.ds(nrows - 16, 16)]
                for r16 in range(16 - rem, 16):
                    r = nrows - 16 + r16
                    vs = [node_v[r, pl.ds(j * 16, 16)]
                          for j in range(HC // 16)]
                    seg = ids.at[jnp.full((16,), r16, jnp.int32)].get(
                        mode="promise_in_bounds")
                    if p == 0:
                        ev = edge_v[r // 8, pl.ds((r % 8) * 16, 16)]
                        erow = lax.shift_right_logical(seg, 2)
                        ecol = lax.shift_left(jnp.bitwise_and(seg, 3), 5)
                    for j in range(HC // 16):
                        plsc.addupdate_scatter(acc_n,
                                               [seg, iota16 + (j * 16)],
                                               vs[j])
                    if p == 0:
                        plsc.addupdate_scatter(acc_ec, [erow, ecol + iota16],
                                               ev)
                        plsc.addupdate_scatter(acc_ec,
                                               [erow, ecol + 16 + iota16],
                                               ones16)

        def block(b, carry):
            block_body(base + b * BLK, idx_v, BLK)
            return carry

        lax.fori_loop(0, FULL_BLOCKS, block, 0)

        @pl.when(wid < _R)
        def _tail_big():
            block_body(toff, idx_tb, TAIL_BIG)

        @pl.when(wid == NW - 1)
        def _tail_extra():
            block_body(EXTRA_OFF, idx_ts, EXTRA)

        pltpu.sync_copy(acc_n, node_out.at[wid, :, pl.ds(p * HC, HC)])
        if p == 0:
            pltpu.sync_copy(acc_ec, ec_out.at[wid])


def _segment_sums(node_attr_prime, edge_out_bar, batch):
    mesh = plsc.VectorSubcoreMesh(core_axis_name="c", subcore_axis_name="s",
                                  num_cores=NC, num_subcores=NS)
    f = pl.kernel(
        _seg_sum_kernel,
        out_type=[
            jax.ShapeDtypeStruct((NW, G, DN), jnp.float32),
            jax.ShapeDtypeStruct((NW, G // 4, HC), jnp.float32),
        ],
        mesh=mesh,
        scratch_types=[
            pltpu.VMEM((BLK, HC), jnp.float32),
            pltpu.VMEM((BLK // 8, HC), jnp.float32),
            pltpu.VMEM((BLK,), jnp.int32),
            pltpu.VMEM((TAIL_BIG,), jnp.int32),
            pltpu.VMEM((EXTRA,), jnp.int32),
            pltpu.VMEM((G, HC), jnp.float32),
            pltpu.VMEM((G // 4, HC), jnp.float32),
        ],
        compiler_params=pltpu.CompilerParams(needs_layout_passes=False),
    )
    edge_folded = edge_out_bar.reshape(N // 8, 8 * DE)
    return f(node_attr_prime, edge_folded, batch)


def _mlp_kernel(u_ref, np_ref, ec_ref, w1_ref, b1_ref, w2_ref,
                b2_ref, out_ref):
    nsum = jnp.sum(np_ref[...], axis=0)
    ec = jnp.sum(ec_ref[...], axis=0)
    esum = ec[:, 0:DE]
    cnt = ec[:, DE:2 * DE]
    inv = 1.0 / jnp.maximum(cnt[:, 0:1], 1.0)
    nmean = nsum * inv
    emean = esum * inv
    h = jnp.dot(u_ref[...], w1_ref[0:32, :],
                preferred_element_type=jnp.float32)
    h = h + jnp.dot(nmean, w1_ref[32:288, :],
                    preferred_element_type=jnp.float32)
    h = h + jnp.dot(emean, w1_ref[288:304, :],
                    preferred_element_type=jnp.float32)
    h = jnp.maximum(h + b1_ref[...], 0.0)
    o = jnp.dot(h, w2_ref[...], preferred_element_type=jnp.float32)
    out_ref[...] = jnp.maximum(o + b2_ref[...], 0.0)


def kernel(node_attr_prime, edge_out_bar, u, batch, W1, b1, W2, b2):
    node_part, ec_part = _segment_sums(node_attr_prime, edge_out_bar, batch)
    ec_part = ec_part.reshape(NW, G, 2 * DE)
    return pl.pallas_call(
        _mlp_kernel,
        out_shape=jax.ShapeDtypeStruct((G, 1), jnp.float32),
    )(u, node_part, ec_part, W1, b1.reshape(1, -1), W2,
      b2.reshape(1, 1))
